# baseline (speedup 1.0000x reference)
import jax
import jax.numpy as jnp
from jax.experimental import pallas as pl


def _aidw_body(src_ref, tar_ref, mask_ref, feat_ref, lin_ref, out_ref):
    diff = src_ref[0] - tar_ref[0]
    d2 = jnp.sum(diff * diff, axis=0, keepdims=True)
    sc = jnp.where(mask_ref[0] != 0.0, 1.0 / d2, 0.0)
    w = sc / jnp.sum(sc)
    out_ref[0] = jnp.dot(feat_ref[0] * w, lin_ref[...],
                         preferred_element_type=jnp.float32)


def kernel(features, src_locs, tar_loc, src_masks, linear):
    B, L, S = features.shape
    O = linear.shape[1]
    src_t = jnp.transpose(src_locs, (0, 2, 1))
    tar_b = tar_loc[:, :, None]
    mask_f = src_masks.astype(jnp.float32)[:, None, :]

    TL = 1024
    return pl.pallas_call(
        _aidw_body,
        grid=(B, L // TL),
        in_specs=[
            pl.BlockSpec((1, 2, S), lambda b, l: (b, 0, 0)),
            pl.BlockSpec((1, 2, 1), lambda b, l: (b, 0, 0)),
            pl.BlockSpec((1, 1, S), lambda b, l: (b, 0, 0)),
            pl.BlockSpec((1, TL, S), lambda b, l: (b, l, 0)),
            pl.BlockSpec((S, O), lambda b, l: (0, 0)),
        ],
        out_specs=pl.BlockSpec((1, TL, O), lambda b, l: (b, l, 0)),
        out_shape=jax.ShapeDtypeStruct((B, L, O), jnp.float32),
    )(src_t, tar_b, mask_f, features, linear)


# speedup vs baseline: 1.7653x; 1.7653x over previous
import jax
import jax.numpy as jnp
from jax.experimental import pallas as pl


def _aidw_body(src_ref, tar_ref, mask_ref, feat_ref, lin_ref, out_ref):
    G = feat_ref.shape[0]
    diff = src_ref[...] - tar_ref[...]
    d2 = jnp.sum(diff * diff, axis=1, keepdims=True)
    sc = jnp.where(mask_ref[...] != 0.0, 1.0 / d2, 0.0)
    w = sc / jnp.sum(sc, axis=2, keepdims=True)
    scaled = feat_ref[...] * w
    flat = scaled.reshape(G * scaled.shape[1], scaled.shape[2])
    out = jnp.dot(flat, lin_ref[...], preferred_element_type=jnp.float32)
    out_ref[...] = out.reshape(G, scaled.shape[1], out.shape[1])


def kernel(features, src_locs, tar_loc, src_masks, linear):
    B, L, S = features.shape
    O = linear.shape[1]
    src_t = jnp.transpose(src_locs, (0, 2, 1))
    tar_b = tar_loc[:, :, None]
    mask_f = src_masks.astype(jnp.float32)[:, None, :]

    G = 4
    return pl.pallas_call(
        _aidw_body,
        grid=(B // G,),
        in_specs=[
            pl.BlockSpec((G, 2, S), lambda b: (b, 0, 0)),
            pl.BlockSpec((G, 2, 1), lambda b: (b, 0, 0)),
            pl.BlockSpec((G, 1, S), lambda b: (b, 0, 0)),
            pl.BlockSpec((G, L, S), lambda b: (b, 0, 0)),
            pl.BlockSpec((S, O), lambda b: (0, 0)),
        ],
        out_specs=pl.BlockSpec((G, L, O), lambda b: (b, 0, 0)),
        out_shape=jax.ShapeDtypeStruct((B, L, O), jnp.float32),
    )(src_t, tar_b, mask_f, features, linear)


# speedup vs baseline: 1.7699x; 1.0026x over previous
import jax
import jax.numpy as jnp
from jax.experimental import pallas as pl


def _aidw_body(src_ref, tar_ref, mask_ref, feat_ref, lin_ref, out_ref):
    G = feat_ref.shape[0]
    diff = src_ref[...] - tar_ref[...]
    d2 = jnp.sum(diff * diff, axis=1, keepdims=True)
    sc = jnp.where(mask_ref[...] != 0.0, 1.0 / d2, 0.0)
    w = sc / jnp.sum(sc, axis=2, keepdims=True)
    scaled = feat_ref[...] * w
    flat = scaled.reshape(G * scaled.shape[1], scaled.shape[2])
    out = jnp.dot(flat, lin_ref[...], preferred_element_type=jnp.float32)
    out_ref[...] = out.reshape(G, scaled.shape[1], out.shape[1])


def kernel(features, src_locs, tar_loc, src_masks, linear):
    B, L, S = features.shape
    O = linear.shape[1]
    src_t = jnp.transpose(src_locs, (0, 2, 1))
    tar_b = tar_loc[:, :, None]
    mask_f = src_masks.astype(jnp.float32)[:, None, :]

    G = 8
    TL = 2048
    return pl.pallas_call(
        _aidw_body,
        grid=(B // G, L // TL),
        in_specs=[
            pl.BlockSpec((G, 2, S), lambda b, l: (b, 0, 0)),
            pl.BlockSpec((G, 2, 1), lambda b, l: (b, 0, 0)),
            pl.BlockSpec((G, 1, S), lambda b, l: (b, 0, 0)),
            pl.BlockSpec((G, TL, S), lambda b, l: (b, l, 0)),
            pl.BlockSpec((S, O), lambda b, l: (0, 0)),
        ],
        out_specs=pl.BlockSpec((G, TL, O), lambda b, l: (b, l, 0)),
        out_shape=jax.ShapeDtypeStruct((B, L, O), jnp.float32),
    )(src_t, tar_b, mask_f, features, linear)


# speedup vs baseline: 1.7717x; 1.0010x over previous
import jax
import jax.numpy as jnp
from jax.experimental import pallas as pl


def _aidw_body(src_ref, tar_ref, mask_ref, feat_ref, lin_ref, out_ref):
    G = feat_ref.shape[0]
    diff = src_ref[...] - tar_ref[...]
    d2 = jnp.sum(diff * diff, axis=1, keepdims=True)
    sc = jnp.where(mask_ref[...] != 0.0, 1.0 / d2, 0.0)
    w = sc / jnp.sum(sc, axis=2, keepdims=True)
    scaled = feat_ref[...] * w
    flat = scaled.reshape(G * scaled.shape[1], scaled.shape[2])
    out = jnp.dot(flat, lin_ref[...], preferred_element_type=jnp.float32)
    out_ref[...] = out.reshape(G, scaled.shape[1], out.shape[1])


def kernel(features, src_locs, tar_loc, src_masks, linear):
    B, L, S = features.shape
    O = linear.shape[1]
    src_t = jnp.transpose(src_locs, (0, 2, 1))
    tar_b = tar_loc[:, :, None]
    mask_f = src_masks.astype(jnp.float32)[:, None, :]

    G = 16
    TL = 1024
    return pl.pallas_call(
        _aidw_body,
        grid=(B // G, L // TL),
        in_specs=[
            pl.BlockSpec((G, 2, S), lambda b, l: (b, 0, 0)),
            pl.BlockSpec((G, 2, 1), lambda b, l: (b, 0, 0)),
            pl.BlockSpec((G, 1, S), lambda b, l: (b, 0, 0)),
            pl.BlockSpec((G, TL, S), lambda b, l: (b, l, 0)),
            pl.BlockSpec((S, O), lambda b, l: (0, 0)),
        ],
        out_specs=pl.BlockSpec((G, TL, O), lambda b, l: (b, l, 0)),
        out_shape=jax.ShapeDtypeStruct((B, L, O), jnp.float32),
    )(src_t, tar_b, mask_f, features, linear)
